# baseline (speedup 1.0000x reference)
import numpy as np
import jax
import jax.numpy as jnp
from jax.experimental import pallas as pl
from jax.experimental.pallas import tpu as pltpu

_DIM = 256
_WD = 8
_T = 16384
_NSEG = 16
_BT = 2048
_G = _T // _BT
_NF = _DIM // 2


def _invf():
    f2 = jax.lax.broadcasted_iota(jnp.int32, (1, _NF), 1).astype(
        jnp.float32) * 2.0
    return jnp.exp(f2 * (-np.log(10000.0) / _DIM))


def _enc_kernel(x_ref, ball_ref, wk_ref, wv_ref,
                out_ref, sr_ref, cr_ref, cb_ref, sb_ref,
                wkv_ref, we_ref, wo_ref, won_ref):
    i = pl.program_id(0)

    @pl.when(i == 0)
    def _init():
        ball = ball_ref[:, 0, :]
        row = jax.lax.broadcasted_iota(jnp.int32, (_NSEG, 1), 0)
        nacc = jnp.zeros((_NSEG, 1), jnp.float32)
        for b in range(_NSEG):
            cnt_b = jnp.sum((ball == b).astype(jnp.int32))
            nacc = nacc + jnp.where(row == b,
                                    cnt_b.astype(jnp.float32), 0.0)
        lr = jax.lax.broadcasted_iota(jnp.int32, (_NSEG, _NSEG), 0)
        lc = jax.lax.broadcasted_iota(jnp.int32, (_NSEG, _NSEG), 1)
        L = (lc < lr).astype(jnp.float32)
        oacc = jax.lax.dot_general(L, nacc, (((1,), (0,)), ((), ())),
                                   preferred_element_type=jnp.float32,
                                   precision=jax.lax.Precision.HIGHEST)

        invf = _invf()
        rcol = jax.lax.broadcasted_iota(jnp.int32, (_BT // 4, 1), 0).astype(
            jnp.float32)
        rf = rcol * invf
        s0 = jnp.sin(rf)
        c0 = jnp.cos(rf)
        cq = jnp.cos(invf * float(_BT // 4))
        sq = jnp.sin(invf * float(_BT // 4))
        s1 = jnp.concatenate([s0, s0 * cq + c0 * sq], axis=0)
        c1 = jnp.concatenate([c0, c0 * cq - s0 * sq], axis=0)
        ch = jnp.cos(invf * float(_BT // 2))
        sh = jnp.sin(invf * float(_BT // 2))
        sr_ref[0:_BT // 2, :] = s1
        cr_ref[0:_BT // 2, :] = c1
        sr_ref[_BT // 2:_BT, :] = s1 * ch + c1 * sh
        cr_ref[_BT // 2:_BT, :] = c1 * ch - s1 * sh
        offf = oacc * invf
        cb_ref[:] = jnp.cos(offf)
        sb_ref[:] = jnp.sin(offf)

        wkv = jnp.concatenate([wk_ref[:], wv_ref[:]], axis=0)
        wkv_ref[:] = wkv
        ec2 = jax.lax.broadcasted_iota(jnp.int32, (_DIM, _NF), 0)
        fc = jax.lax.broadcasted_iota(jnp.int32, (_DIM, _NF), 1)
        sel_e = (ec2 == 2 * fc).astype(jnp.float32)
        sel_o = (ec2 == 2 * fc + 1).astype(jnp.float32)
        we_ref[:] = jax.lax.dot_general(wkv, sel_e, (((1,), (0,)), ((), ())),
                                        preferred_element_type=jnp.float32,
                                        precision=jax.lax.Precision.HIGHEST)
        wo = jax.lax.dot_general(wkv, sel_o, (((1,), (0,)), ((), ())),
                                 preferred_element_type=jnp.float32,
                                 precision=jax.lax.Precision.HIGHEST)
        wo_ref[:] = wo
        won_ref[:] = -wo

        out_ref[:] = jnp.zeros_like(out_ref)
        out_ref[:, _WD * _WD:_WD * _WD + 1] = nacc

    brow = ball_ref[i]
    seg = jax.lax.broadcasted_iota(jnp.int32, (_NSEG, 1), 0)
    S2r = (brow == seg).astype(jnp.float32)

    invf = _invf()
    ci = invf * jax.lax.convert_element_type(i * _BT, jnp.float32)
    cci = jnp.cos(ci)
    sci = jnp.sin(ci)
    cos_phi = cci * cb_ref[:] + sci * sb_ref[:]
    sin_phi = sci * cb_ref[:] - cci * sb_ref[:]
    phi_cat = jnp.concatenate([cos_phi, sin_phi], axis=1)

    g = jax.lax.dot_general(S2r, phi_cat, (((0,), (0,)), ((), ())),
                            preferred_element_type=jnp.float32)
    c_tok = g[:, 0:_NF]
    s_tok = g[:, _NF:2 * _NF]

    sr = sr_ref[:]
    cr = cr_ref[:]
    kv = (jax.lax.dot_general(x_ref[:], wkv_ref[:], (((1,), (1,)), ((), ())),
                              preferred_element_type=jnp.float32)
          + jax.lax.dot_general(sr * c_tok, we_ref[:],
                                (((1,), (1,)), ((), ())),
                                preferred_element_type=jnp.float32)
          + jax.lax.dot_general(cr * s_tok, we_ref[:],
                                (((1,), (1,)), ((), ())),
                                preferred_element_type=jnp.float32)
          + jax.lax.dot_general(cr * c_tok, wo_ref[:],
                                (((1,), (1,)), ((), ())),
                                preferred_element_type=jnp.float32)
          + jax.lax.dot_general(sr * s_tok, won_ref[:],
                                (((1,), (1,)), ((), ())),
                                preferred_element_type=jnp.float32))
    k = kv[:, 0:_WD]
    v = kv[:, _WD:2 * _WD]

    er = jax.lax.broadcasted_iota(jnp.int32, (_WD, _WD * _WD), 0)
    ec = jax.lax.broadcasted_iota(jnp.int32, (_WD, _WD * _WD), 1)
    E = ((ec // _WD) == er).astype(jnp.float32)
    F = ((ec % _WD) == er).astype(jnp.float32)
    vrep = jax.lax.dot_general(v, E, (((1,), (0,)), ((), ())),
                               preferred_element_type=jnp.float32)
    ktile = jax.lax.dot_general(k, F, (((1,), (0,)), ((), ())),
                                preferred_element_type=jnp.float32)
    M = vrep * ktile

    zp = jax.lax.dot_general(S2r, M, (((1,), (0,)), ((), ())),
                             preferred_element_type=jnp.float32)
    out_ref[:, 0:_WD * _WD] = out_ref[:, 0:_WD * _WD] + zp


def kernel(x, batch, Wk, Wv):
    b32 = batch.astype(jnp.int32)
    ball = b32.reshape(_G, 1, _BT)

    out = pl.pallas_call(
        _enc_kernel,
        grid=(_G,),
        in_specs=[
            pl.BlockSpec((_BT, _DIM), lambda i: (i, 0)),
            pl.BlockSpec((_G, 1, _BT), lambda i: (0, 0, 0)),
            pl.BlockSpec((_WD, _DIM), lambda i: (0, 0)),
            pl.BlockSpec((_WD, _DIM), lambda i: (0, 0)),
        ],
        out_specs=pl.BlockSpec((_NSEG, _WD * _WD + 1), lambda i: (0, 0)),
        out_shape=jax.ShapeDtypeStruct((_NSEG, _WD * _WD + 1), jnp.float32),
        scratch_shapes=[
            pltpu.VMEM((_BT, _NF), jnp.float32),
            pltpu.VMEM((_BT, _NF), jnp.float32),
            pltpu.VMEM((_NSEG, _NF), jnp.float32),
            pltpu.VMEM((_NSEG, _NF), jnp.float32),
            pltpu.VMEM((2 * _WD, _DIM), jnp.float32),
            pltpu.VMEM((2 * _WD, _NF), jnp.float32),
            pltpu.VMEM((2 * _WD, _NF), jnp.float32),
            pltpu.VMEM((2 * _WD, _NF), jnp.float32),
        ],
        compiler_params=pltpu.CompilerParams(
            dimension_semantics=("arbitrary",)),
    )(x, ball, Wk, Wv)
    return out


# speedup vs baseline: 1.1813x; 1.1813x over previous
import numpy as np
import jax
import jax.numpy as jnp
from jax.experimental import pallas as pl
from jax.experimental.pallas import tpu as pltpu

_DIM = 256
_WD = 8
_T = 16384
_NSEG = 16
_BT = 2048
_G = _T // _BT
_NF = _DIM // 2


def _invf():
    f2 = jax.lax.broadcasted_iota(jnp.int32, (1, _NF), 1).astype(
        jnp.float32) * 2.0
    return jnp.exp(f2 * (-np.log(10000.0) / _DIM))


def _enc_kernel(x_ref, ball_ref, wk_ref, wv_ref,
                out_ref, sr_ref, cr_ref, cb_ref, sb_ref,
                wkv_ref, we_ref, wo_ref):
    i = pl.program_id(0)

    @pl.when(i == 0)
    def _init():
        ball = ball_ref[:, 0, :]
        row = jax.lax.broadcasted_iota(jnp.int32, (_NSEG, 1), 0)
        nacc = jnp.zeros((_NSEG, 1), jnp.float32)
        for b in range(_NSEG):
            cnt_b = jnp.sum((ball == b).astype(jnp.int32))
            nacc = nacc + jnp.where(row == b,
                                    cnt_b.astype(jnp.float32), 0.0)
        lr = jax.lax.broadcasted_iota(jnp.int32, (_NSEG, _NSEG), 0)
        lc = jax.lax.broadcasted_iota(jnp.int32, (_NSEG, _NSEG), 1)
        L = (lc < lr).astype(jnp.float32)
        oacc = jax.lax.dot_general(L, nacc, (((1,), (0,)), ((), ())),
                                   preferred_element_type=jnp.float32,
                                   precision=jax.lax.Precision.HIGHEST)

        invf = _invf()
        rcol = jax.lax.broadcasted_iota(jnp.int32, (_BT // 4, 1), 0).astype(
            jnp.float32)
        rf = rcol * invf
        s0 = jnp.sin(rf)
        c0 = jnp.cos(rf)
        cq = jnp.cos(invf * float(_BT // 4))
        sq = jnp.sin(invf * float(_BT // 4))
        s1 = jnp.concatenate([s0, s0 * cq + c0 * sq], axis=0)
        c1 = jnp.concatenate([c0, c0 * cq - s0 * sq], axis=0)
        ch = jnp.cos(invf * float(_BT // 2))
        sh = jnp.sin(invf * float(_BT // 2))
        sr_ref[0:_BT // 2, :] = s1
        cr_ref[0:_BT // 2, :] = c1
        sr_ref[_BT // 2:_BT, :] = s1 * ch + c1 * sh
        cr_ref[_BT // 2:_BT, :] = c1 * ch - s1 * sh
        offf = oacc * invf
        cb_ref[:] = jnp.cos(offf)
        sb_ref[:] = jnp.sin(offf)

        wkv = jnp.concatenate([wk_ref[:], wv_ref[:]], axis=0)
        wkv_ref[:] = wkv
        ec2 = jax.lax.broadcasted_iota(jnp.int32, (_DIM, _NF), 0)
        fc = jax.lax.broadcasted_iota(jnp.int32, (_DIM, _NF), 1)
        sel_e = (ec2 == 2 * fc).astype(jnp.float32)
        sel_o = (ec2 == 2 * fc + 1).astype(jnp.float32)
        we_ref[:] = jax.lax.dot_general(wkv, sel_e, (((1,), (0,)), ((), ())),
                                        preferred_element_type=jnp.float32,
                                        precision=jax.lax.Precision.HIGHEST)
        wo_ref[:] = jax.lax.dot_general(wkv, sel_o, (((1,), (0,)), ((), ())),
                                        preferred_element_type=jnp.float32,
                                        precision=jax.lax.Precision.HIGHEST)

        out_ref[:] = jnp.zeros_like(out_ref)
        out_ref[:, _WD * _WD:_WD * _WD + 1] = nacc

    brow = ball_ref[i]
    seg = jax.lax.broadcasted_iota(jnp.int32, (_NSEG, 1), 0)
    S2r = (brow == seg).astype(jnp.float32)

    invf = _invf()
    ci = invf * jax.lax.convert_element_type(i * _BT, jnp.float32)
    cci = jnp.cos(ci)
    sci = jnp.sin(ci)
    cos_phi = cci * cb_ref[:] + sci * sb_ref[:]
    sin_phi = sci * cb_ref[:] - cci * sb_ref[:]
    phi_cat = jnp.concatenate([cos_phi, sin_phi], axis=1)

    g = jax.lax.dot_general(S2r, phi_cat, (((0,), (0,)), ((), ())),
                            preferred_element_type=jnp.float32)
    c_tok = g[:, 0:_NF]
    s_tok = g[:, _NF:2 * _NF]

    sr = sr_ref[:]
    cr = cr_ref[:]
    sin_ang = sr * c_tok + cr * s_tok
    cos_ang = cr * c_tok - sr * s_tok

    kv = (jax.lax.dot_general(x_ref[:], wkv_ref[:], (((1,), (1,)), ((), ())),
                              preferred_element_type=jnp.float32)
          + jax.lax.dot_general(sin_ang, we_ref[:],
                                (((1,), (1,)), ((), ())),
                                preferred_element_type=jnp.float32)
          + jax.lax.dot_general(cos_ang, wo_ref[:],
                                (((1,), (1,)), ((), ())),
                                preferred_element_type=jnp.float32))
    gr = jax.lax.broadcasted_iota(jnp.int32, (2 * _WD, 2 * _WD * _WD), 0)
    gc = jax.lax.broadcasted_iota(jnp.int32, (2 * _WD, 2 * _WD * _WD), 1)
    vpart = (gr >= _WD) & (gc < _WD * _WD) & (gc // _WD == gr - _WD)
    kpart = (gr < _WD) & (gc >= _WD * _WD) & ((gc - _WD * _WD) % _WD == gr)
    Gbd = (vpart | kpart).astype(jnp.float32)
    h = jax.lax.dot_general(kv, Gbd, (((1,), (0,)), ((), ())),
                            preferred_element_type=jnp.float32)
    M = h[:, 0:_WD * _WD] * h[:, _WD * _WD:2 * _WD * _WD]

    zp = jax.lax.dot_general(S2r, M, (((1,), (0,)), ((), ())),
                             preferred_element_type=jnp.float32)
    out_ref[:, 0:_WD * _WD] = out_ref[:, 0:_WD * _WD] + zp


def kernel(x, batch, Wk, Wv):
    b32 = batch.astype(jnp.int32)
    ball = b32.reshape(_G, 1, _BT)

    out = pl.pallas_call(
        _enc_kernel,
        grid=(_G,),
        in_specs=[
            pl.BlockSpec((_BT, _DIM), lambda i: (i, 0)),
            pl.BlockSpec((_G, 1, _BT), lambda i: (0, 0, 0)),
            pl.BlockSpec((_WD, _DIM), lambda i: (0, 0)),
            pl.BlockSpec((_WD, _DIM), lambda i: (0, 0)),
        ],
        out_specs=pl.BlockSpec((_NSEG, _WD * _WD + 1), lambda i: (0, 0)),
        out_shape=jax.ShapeDtypeStruct((_NSEG, _WD * _WD + 1), jnp.float32),
        scratch_shapes=[
            pltpu.VMEM((_BT, _NF), jnp.float32),
            pltpu.VMEM((_BT, _NF), jnp.float32),
            pltpu.VMEM((_NSEG, _NF), jnp.float32),
            pltpu.VMEM((_NSEG, _NF), jnp.float32),
            pltpu.VMEM((2 * _WD, _DIM), jnp.float32),
            pltpu.VMEM((2 * _WD, _NF), jnp.float32),
            pltpu.VMEM((2 * _WD, _NF), jnp.float32),
        ],
        compiler_params=pltpu.CompilerParams(
            dimension_semantics=("arbitrary",)),
    )(x, ball, Wk, Wv)
    return out
